# baseline (speedup 1.0000x reference)
import jax
import jax.numpy as jnp
from jax import lax
from jax.experimental import pallas as pl
from jax.experimental.pallas import tpu as pltpu
from jax.experimental.pallas import tpu_sc as plsc

NC = 2
NS = 16
B = 256
C, H, W = 4, 64, 64
P = C * H
PBLK = 64
PA = 64
TPAD = 1024


def _gather_body(t_hbm, ac_hbm, am_hbm, a_out, am_out, t_v, a_v, am_v, csem):
    wid = lax.axis_index("s")

    @pl.when(wid == 0)
    def _():
        pltpu.sync_copy(t_hbm, t_v)
        ccps = []
        for h in range(2):
            sl = pl.ds(h * 128, 128)
            ccps.append(pltpu.async_copy(
                ac_hbm.at[t_v.at[sl]], a_v.at[sl], csem))
            ccps.append(pltpu.async_copy(
                am_hbm.at[t_v.at[sl]], am_v.at[sl], csem))
        for cp in ccps:
            cp.wait()
        cpo = pltpu.async_copy(a_v, a_out, csem)
        cpm = pltpu.async_copy(am_v, am_out, csem)
        cpo.wait()
        cpm.wait()


def _sc_gather(t32, ac, am):
    mesh = plsc.VectorSubcoreMesh(
        core_axis_name="c", subcore_axis_name="s",
        num_cores=1, num_subcores=NS)
    f = pl.kernel(
        _gather_body,
        out_type=(jax.ShapeDtypeStruct((B,), jnp.float32),
                  jax.ShapeDtypeStruct((B,), jnp.float32)),
        mesh=mesh,
        scratch_types=[
            pltpu.VMEM((B,), jnp.int32),
            pltpu.VMEM((B,), jnp.float32),
            pltpu.VMEM((B,), jnp.float32),
            pltpu.SemaphoreType.DMA,
        ],
    )
    return f(t32, ac, am)


def _tcg_body(t_ref, tbl_ref, a_ref, am_ref):
    iot = lax.broadcasted_iota(jnp.int32, (TPAD, B), 0)
    oh = jnp.where(iot == t_ref[...], 1.0, 0.0).astype(jnp.float32)
    cf = jnp.dot(tbl_ref[...], oh, preferred_element_type=jnp.float32,
                 precision=lax.Precision.HIGHEST)
    a_ref[...] = cf[0:1, :].reshape(1, 1, B)
    am_ref[...] = cf[1:2, :].reshape(1, 1, B)


def _tc_gather(t2, tbl2):
    return pl.pallas_call(
        _tcg_body,
        out_shape=(jax.ShapeDtypeStruct((1, 1, B), jnp.float32),
                   jax.ShapeDtypeStruct((1, 1, B), jnp.float32)),
    )(t2, tbl2)


def _dense_a_body(a_ref, am_ref, x_ref, n_ref, o_ref, no_ref):
    n = n_ref[...]
    o_ref[...] = a_ref[...] * x_ref[...] + am_ref[...] * n
    no_ref[...] = n


def _dense_a(a2, am2, x0T, nT):
    blk = pl.BlockSpec((PBLK, W, B), lambda i: (i, 0, 0))
    cblk = pl.BlockSpec((1, 1, B), lambda i: (0, 0, 0))
    return pl.pallas_call(
        _dense_a_body,
        grid=(PA // PBLK,),
        in_specs=[cblk, cblk, blk, blk],
        out_specs=(blk, blk),
        out_shape=(jax.ShapeDtypeStruct((P, W, B), jnp.float32),
                   jax.ShapeDtypeStruct((P, W, B), jnp.float32)),
    )(a2, am2, x0T, nT)


def _dense_b_body(o_in, no_in, a_ref, am_ref, x_ref, n_ref, o_ref, no_ref):
    del o_in, no_in
    n = n_ref[...]
    o_ref[...] = a_ref[...] * x_ref[...] + am_ref[...] * n
    no_ref[...] = n


def _dense_b(outA, noutA, a2, am2, x0T, nT):
    off = PA // PBLK
    blk = pl.BlockSpec((PBLK, W, B), lambda i: (i + off, 0, 0))
    cblk = pl.BlockSpec((1, 1, B), lambda i: (0, 0, 0))
    anyspec = pl.BlockSpec(memory_space=pl.ANY)
    return pl.pallas_call(
        _dense_b_body,
        grid=((P - PA) // PBLK,),
        in_specs=[anyspec, anyspec, cblk, cblk, blk, blk],
        out_specs=(blk, blk),
        out_shape=(jax.ShapeDtypeStruct((P, W, B), jnp.float32),
                   jax.ShapeDtypeStruct((P, W, B), jnp.float32)),
        input_output_aliases={0: 0, 1: 1},
    )(outA, noutA, a2, am2, x0T, nT)


@jax.jit
def _run(x0, t32, noise, ac, am):
    x0T = x0.transpose(1, 2, 3, 0).reshape(P, W, B)
    nT = noise.transpose(1, 2, 3, 0).reshape(P, W, B)
    a_sc, am_sc = _sc_gather(t32, ac, am)
    tbl2 = jnp.stack([jnp.pad(ac, (0, TPAD - ac.shape[0])),
                      jnp.pad(am, (0, TPAD - am.shape[0]))])
    a_tc, am_tc = _tc_gather(t32.reshape(1, B), tbl2)
    outA, noutA = _dense_a(a_tc, am_tc, x0T, nT)
    outT, noutT = _dense_b(outA, noutA, a_sc.reshape(1, 1, B),
                           am_sc.reshape(1, 1, B), x0T, nT)
    out = outT.reshape(C, H, W, B).transpose(3, 0, 1, 2)
    nout = noutT.reshape(C, H, W, B).transpose(3, 0, 1, 2)
    return out, nout


def kernel(x0, t, noise, sqrt_ac, sqrt_1mac):
    return _run(x0, t.astype(jnp.int32), noise, sqrt_ac, sqrt_1mac)


# speedup vs baseline: 1.0411x; 1.0411x over previous
import jax
import jax.numpy as jnp
from jax import lax
from jax.experimental import pallas as pl
from jax.experimental.pallas import tpu as pltpu
from jax.experimental.pallas import tpu_sc as plsc

NC = 2
NS = 16
B = 256
C, H, W = 4, 64, 64
P = C * H
PBLK_A = 64
PBLK_B = 64
PA = 64
T_LEN = 1000


def _gather_body(t_hbm, ac_hbm, am_hbm, a_out, am_out, t_v, a_v, am_v, csem):
    wid = lax.axis_index("s")

    @pl.when(wid == 0)
    def _():
        pltpu.sync_copy(t_hbm, t_v)
        ccps = []
        for h in range(2):
            sl = pl.ds(h * 128, 128)
            ccps.append(pltpu.async_copy(
                ac_hbm.at[t_v.at[sl]], a_v.at[sl], csem))
            ccps.append(pltpu.async_copy(
                am_hbm.at[t_v.at[sl]], am_v.at[sl], csem))
        for cp in ccps:
            cp.wait()
        cpo = pltpu.async_copy(a_v, a_out, csem)
        cpm = pltpu.async_copy(am_v, am_out, csem)
        cpo.wait()
        cpm.wait()


def _sc_gather(t32, ac, am):
    mesh = plsc.VectorSubcoreMesh(
        core_axis_name="c", subcore_axis_name="s",
        num_cores=1, num_subcores=NS)
    f = pl.kernel(
        _gather_body,
        out_type=(jax.ShapeDtypeStruct((B,), jnp.float32),
                  jax.ShapeDtypeStruct((B,), jnp.float32)),
        mesh=mesh,
        scratch_types=[
            pltpu.VMEM((B,), jnp.int32),
            pltpu.VMEM((B,), jnp.float32),
            pltpu.VMEM((B,), jnp.float32),
            pltpu.SemaphoreType.DMA,
        ],
    )
    return f(t32, ac, am)


def _tcg_body(t_ref, ac_ref, am_ref, a_ref, am_out_ref):
    iot = lax.broadcasted_iota(jnp.int32, (T_LEN, B), 0)
    oh = jnp.where(iot == t_ref[...], 1.0, 0.0).astype(jnp.float32)
    a_ref[...] = jnp.dot(ac_ref[...], oh, preferred_element_type=jnp.float32,
                         precision=lax.Precision.HIGHEST).reshape(1, 1, B)
    am_out_ref[...] = jnp.dot(am_ref[...], oh,
                              preferred_element_type=jnp.float32,
                              precision=lax.Precision.HIGHEST).reshape(1, 1, B)


def _tc_gather(t2, ac2, am2):
    return pl.pallas_call(
        _tcg_body,
        out_shape=(jax.ShapeDtypeStruct((1, 1, B), jnp.float32),
                   jax.ShapeDtypeStruct((1, 1, B), jnp.float32)),
    )(t2, ac2, am2)


def _dense_a_body(a_ref, am_ref, x_ref, n_ref, o_ref, no_ref):
    n = n_ref[...]
    o_ref[...] = a_ref[...] * x_ref[...] + am_ref[...] * n
    no_ref[...] = n


def _dense_a(a2, am2, x0T, nT):
    blk = pl.BlockSpec((PBLK_A, W, B), lambda i: (i, 0, 0))
    cblk = pl.BlockSpec((1, 1, B), lambda i: (0, 0, 0))
    return pl.pallas_call(
        _dense_a_body,
        grid=(PA // PBLK_A,),
        in_specs=[cblk, cblk, blk, blk],
        out_specs=(blk, blk),
        out_shape=(jax.ShapeDtypeStruct((P, W, B), jnp.float32),
                   jax.ShapeDtypeStruct((P, W, B), jnp.float32)),
    )(a2, am2, x0T, nT)


def _dense_b_body(o_in, no_in, a_ref, am_ref, x_ref, n_ref, o_ref, no_ref):
    del o_in, no_in
    n = n_ref[...]
    o_ref[...] = a_ref[...] * x_ref[...] + am_ref[...] * n
    no_ref[...] = n


def _dense_b(outA, noutA, a2, am2, x0T, nT):
    off = PA // PBLK_B
    blk = pl.BlockSpec((PBLK_B, W, B), lambda i: (i + off, 0, 0))
    cblk = pl.BlockSpec((1, 1, B), lambda i: (0, 0, 0))
    anyspec = pl.BlockSpec(memory_space=pl.ANY)
    return pl.pallas_call(
        _dense_b_body,
        grid=((P - PA) // PBLK_B,),
        in_specs=[anyspec, anyspec, cblk, cblk, blk, blk],
        out_specs=(blk, blk),
        out_shape=(jax.ShapeDtypeStruct((P, W, B), jnp.float32),
                   jax.ShapeDtypeStruct((P, W, B), jnp.float32)),
        input_output_aliases={0: 0, 1: 1},
    )(outA, noutA, a2, am2, x0T, nT)


@jax.jit
def _run(x0, t32, noise, ac, am):
    x0T = x0.transpose(1, 2, 3, 0).reshape(P, W, B)
    nT = noise.transpose(1, 2, 3, 0).reshape(P, W, B)
    a_sc, am_sc = _sc_gather(t32, ac, am)
    a_tc, am_tc = _tc_gather(t32.reshape(1, B), ac.reshape(1, T_LEN),
                             am.reshape(1, T_LEN))
    outA, noutA = _dense_a(a_tc, am_tc, x0T, nT)
    outT, noutT = _dense_b(outA, noutA, a_sc.reshape(1, 1, B),
                           am_sc.reshape(1, 1, B), x0T, nT)
    out = outT.reshape(C, H, W, B).transpose(3, 0, 1, 2)
    nout = noutT.reshape(C, H, W, B).transpose(3, 0, 1, 2)
    return out, nout


def kernel(x0, t, noise, sqrt_ac, sqrt_1mac):
    return _run(x0, t.astype(jnp.int32), noise, sqrt_ac, sqrt_1mac)
